# baseline (speedup 1.0000x reference)
import jax
import jax.numpy as jnp
from jax import lax
from jax.experimental import pallas as pl
from jax.experimental.pallas import tpu as pltpu
from jax.experimental.pallas import tpu_sc as plsc

_NC = 2
_NS = 16
_NW = _NC * _NS
_L = 16

_BN_EPS = 1e-5
_TM = 512


def _gather_sum7_sc(table, idx, n_out, scale):
    T, C = table.shape
    PW = n_out // _NW
    V = 16
    NCH = PW // V
    CL = C // _L

    mesh = plsc.VectorSubcoreMesh(
        core_axis_name="c", subcore_axis_name="s",
        num_cores=_NC, num_subcores=_NS)

    def body(tab_hbm, idx_hbm, out_hbm, idx_v, rows_v, acc_v, sem):
        wid = lax.axis_index("s") * _NC + lax.axis_index("c")
        base = wid * PW

        def chunk(ci, carry):
            rb = base + ci * V
            pltpu.sync_copy(idx_hbm.at[pl.ds(rb * 7, V * 7)], idx_v)
            pltpu.async_copy(tab_hbm.at[idx_v], rows_v, sem).wait()

            def per_row(v, c2):
                for c in range(CL):
                    sl = pl.ds(c * _L, _L)
                    s = rows_v[v * 7, sl]
                    for k in range(1, 7):
                        s = s + rows_v[v * 7 + k, sl]
                    acc_v[v, sl] = s * scale
                return c2

            lax.fori_loop(0, V, per_row, 0)
            pltpu.sync_copy(acc_v, out_hbm.at[pl.ds(rb, V)])
            return carry

        lax.fori_loop(0, NCH, chunk, 0)

    f = pl.kernel(
        body,
        out_type=jax.ShapeDtypeStruct((n_out, C), jnp.float32),
        mesh=mesh,
        scratch_types=[
            pltpu.VMEM((V * 7,), jnp.int32),
            pltpu.VMEM((V * 7, C), jnp.float32),
            pltpu.VMEM((V, C), jnp.float32),
            pltpu.SemaphoreType.DMA,
        ],
    )
    return f(table, idx)


def _zp_matmul_tc(xp, Wz):
    P, C = xp.shape
    K7 = Wz.shape[1]
    G = P // _TM

    def body(x_ref, w_ref, o_ref):
        o_ref[...] = jnp.dot(x_ref[...], w_ref[...],
                             preferred_element_type=jnp.float32)

    return pl.pallas_call(
        body,
        grid=(G,),
        in_specs=[pl.BlockSpec((_TM, C), lambda i: (i, 0)),
                  pl.BlockSpec((C, K7), lambda i: (0, 0))],
        out_specs=pl.BlockSpec((_TM, K7), lambda i: (i, 0)),
        out_shape=jax.ShapeDtypeStruct((P, K7), jnp.float32),
    )(xp, Wz)


def _stats_tc(z, n_valid):
    P, C = z.shape
    G = P // _TM

    def body(z_ref, s_ref):
        i = pl.program_id(0)

        @pl.when(i == 0)
        def _():
            s_ref[...] = jnp.zeros_like(s_ref)

        rows = lax.broadcasted_iota(jnp.int32, (_TM, C), 0) + i * _TM
        zm = jnp.where(rows < n_valid, z_ref[...], 0.0)
        s_ref[0:1, :] += jnp.sum(zm, axis=0, keepdims=True)
        s_ref[1:2, :] += jnp.sum(zm * zm, axis=0, keepdims=True)

    return pl.pallas_call(
        body,
        grid=(G,),
        in_specs=[pl.BlockSpec((_TM, C), lambda i: (i, 0))],
        out_specs=pl.BlockSpec((8, C), lambda i: (0, 0)),
        out_shape=jax.ShapeDtypeStruct((8, C), jnp.float32),
    )(z)


def _final_tc(z, x1p, stats, params, WaT, WbT, n_valid):
    P, C = z.shape
    G = P // _TM
    inv_n = 1.0 / float(n_valid)

    def body(z_ref, x1_ref, s_ref, p_ref, wa_ref, wb_ref, o_ref):
        mean = s_ref[0:1, :] * inv_n
        var = s_ref[1:2, :] * inv_n - mean * mean
        sc = p_ref[0:1, :] * lax.rsqrt(var + _BN_EPS)
        tr = p_ref[1:2, :] - mean * sc
        zn = z_ref[...] * sc + tr
        zn = jnp.where(zn >= 0, zn, 0.2 * zn)
        acc = jnp.dot(zn, wa_ref[...], preferred_element_type=jnp.float32)
        acc = acc + jnp.dot(x1_ref[...], wb_ref[...],
                            preferred_element_type=jnp.float32)
        o_ref[...] = acc + p_ref[2:3, :]

    return pl.pallas_call(
        body,
        grid=(G,),
        in_specs=[pl.BlockSpec((_TM, C), lambda i: (i, 0)),
                  pl.BlockSpec((_TM, C), lambda i: (i, 0)),
                  pl.BlockSpec((8, C), lambda i: (0, 0)),
                  pl.BlockSpec((8, C), lambda i: (0, 0)),
                  pl.BlockSpec((C, C), lambda i: (0, 0)),
                  pl.BlockSpec((C, C), lambda i: (0, 0))],
        out_specs=pl.BlockSpec((_TM, C), lambda i: (i, 0)),
        out_shape=jax.ShapeDtypeStruct((P, C), jnp.float32),
    )(z, x1p, stats, params, WaT, WbT)


def kernel(x, x1, neigh_orders, pool_neigh_orders, W1, b1, gamma, beta, Wc, bc):
    B, C, Nh = x.shape
    CO = x1.shape[1]
    Nl = (Nh + 6) // 4
    R = B * Nl
    P = ((R + _NW * 16 - 1) // (_NW * 16)) * (_NW * 16)

    xT = x.transpose(0, 2, 1).reshape(B * Nh, C)
    x1p = jnp.pad(x1.transpose(0, 2, 1).reshape(R, CO), ((0, P - R), (0, 0)))

    boffs_h = (jnp.arange(B, dtype=jnp.int32) * Nh)[:, None]
    pool_abs = (pool_neigh_orders[: Nl * 7][None, :] + boffs_h).reshape(-1)
    pool_abs = jnp.pad(pool_abs, (0, (P - R) * 7))

    boffs_l = (jnp.arange(B, dtype=jnp.int32) * Nl)[:, None]
    k_off = jnp.tile(jnp.arange(7, dtype=jnp.int32), Nl)[None, :]
    ring_abs = ((neigh_orders[None, :] + boffs_l) * 7 + k_off).reshape(-1)
    ring_abs = jnp.pad(ring_abs, (0, (P - R) * 7))

    Wz = W1.reshape(CO, 7, C).transpose(2, 1, 0).reshape(C, 7 * CO)
    WaT = Wc[:, :CO].T
    WbT = Wc[:, CO:].T
    params = jnp.concatenate(
        [gamma[None, :], beta[None, :], bc[None, :],
         jnp.zeros((5, CO), jnp.float32)], axis=0)

    xp = _gather_sum7_sc(xT, pool_abs, P, 1.0 / 7.0)
    zp = _zp_matmul_tc(xp, Wz)
    z = _gather_sum7_sc(zp.reshape(P * 7, CO), ring_abs, P, 1.0)
    stats = _stats_tc(z, R)
    yT = _final_tc(z, x1p, stats, params, WaT, WbT, R)
    return yT[:R].reshape(B, Nl, CO).transpose(0, 2, 1)


# speedup vs baseline: 1.3934x; 1.3934x over previous
import jax
import jax.numpy as jnp
from jax import lax
from jax.experimental import pallas as pl
from jax.experimental.pallas import tpu as pltpu
from jax.experimental.pallas import tpu_sc as plsc

_NC = 2
_NS = 16
_NW = _NC * _NS
_L = 16

_BN_EPS = 1e-5
_TM = 512


def _gather_sum7_sc(table, idx, n_out):
    T, C = table.shape
    V = 16
    NBUF = 3
    PW = n_out // _NW
    NCH = PW // V
    NGRP = NCH // NBUF
    CL = C // _L

    mesh = plsc.VectorSubcoreMesh(
        core_axis_name="c", subcore_axis_name="s",
        num_cores=_NC, num_subcores=_NS)

    def body(tab_hbm, idx_hbm, out_hbm,
             idx_all, r0, r1, r2, a0, a1, a2, g0, g1, g2, o0, o1, o2):
        rows = (r0, r1, r2)
        acc = (a0, a1, a2)
        semg = (g0, g1, g2)
        semo = (o0, o1, o2)
        wid = lax.axis_index("s") * _NC + lax.axis_index("c")
        base = wid * PW
        pltpu.sync_copy(idx_hbm.at[wid], idx_all)
        for b in range(NBUF):
            pltpu.async_copy(tab_hbm.at[idx_all.at[b]], rows[b], semg[b])

        def group(g, carry):
            for b in range(NBUF):
                ci = g * NBUF + b
                pltpu.make_async_copy(
                    tab_hbm.at[idx_all.at[ci]], rows[b], semg[b]).wait()

                @pl.when(g > 0)
                def _():
                    pltpu.make_async_copy(
                        acc[b],
                        out_hbm.at[pl.ds(base + (ci - NBUF) * V, V)],
                        semo[b]).wait()

                def per_row(v, c2):
                    for c in range(CL):
                        sl = pl.ds(c * _L, _L)
                        s = rows[b][v * 7, sl]
                        for k in range(1, 7):
                            s = s + rows[b][v * 7 + k, sl]
                        acc[b][v, sl] = s
                    return c2

                lax.fori_loop(0, V, per_row, 0)
                pltpu.async_copy(
                    acc[b], out_hbm.at[pl.ds(base + ci * V, V)], semo[b])

                @pl.when(ci + NBUF < NCH)
                def _():
                    pltpu.async_copy(
                        tab_hbm.at[idx_all.at[ci + NBUF]], rows[b], semg[b])
            return carry

        lax.fori_loop(0, NGRP, group, 0)
        for b in range(NBUF):
            ci = NCH - NBUF + b
            pltpu.make_async_copy(
                acc[b], out_hbm.at[pl.ds(base + ci * V, V)], semo[b]).wait()

    f = pl.kernel(
        body,
        out_type=jax.ShapeDtypeStruct((n_out, C), jnp.float32),
        mesh=mesh,
        scratch_types=(
            [pltpu.VMEM((NCH, V * 7), jnp.int32)]
            + [pltpu.VMEM((V * 7, C), jnp.float32)] * 3
            + [pltpu.VMEM((V, C), jnp.float32)] * 3
            + [pltpu.SemaphoreType.DMA] * 6
        ),
    )
    return f(table, idx)


def _zp_matmul_tc(xp, Wz, scale):
    P, C = xp.shape
    K7 = Wz.shape[1]
    G = P // _TM

    def body(x_ref, w_ref, o_ref):
        o_ref[...] = jnp.dot(x_ref[...] * scale, w_ref[...],
                             preferred_element_type=jnp.float32)

    return pl.pallas_call(
        body,
        grid=(G,),
        in_specs=[pl.BlockSpec((_TM, C), lambda i: (i, 0)),
                  pl.BlockSpec((C, K7), lambda i: (0, 0))],
        out_specs=pl.BlockSpec((_TM, K7), lambda i: (i, 0)),
        out_shape=jax.ShapeDtypeStruct((P, K7), jnp.float32),
    )(xp, Wz)


def _stats_tc(z, n_valid):
    P, C = z.shape
    G = P // _TM

    def body(z_ref, s_ref):
        i = pl.program_id(0)

        @pl.when(i == 0)
        def _():
            s_ref[...] = jnp.zeros_like(s_ref)

        rows = lax.broadcasted_iota(jnp.int32, (_TM, C), 0) + i * _TM
        zm = jnp.where(rows < n_valid, z_ref[...], 0.0)
        s_ref[0:1, :] += jnp.sum(zm, axis=0, keepdims=True)
        s_ref[1:2, :] += jnp.sum(zm * zm, axis=0, keepdims=True)

    return pl.pallas_call(
        body,
        grid=(G,),
        in_specs=[pl.BlockSpec((_TM, C), lambda i: (i, 0))],
        out_specs=pl.BlockSpec((8, C), lambda i: (0, 0)),
        out_shape=jax.ShapeDtypeStruct((8, C), jnp.float32),
    )(z)


def _final_tc(z, x1p, stats, params, WaT, WbT, n_valid):
    P, C = z.shape
    G = P // _TM
    inv_n = 1.0 / float(n_valid)

    def body(z_ref, x1_ref, s_ref, p_ref, wa_ref, wb_ref, o_ref):
        mean = s_ref[0:1, :] * inv_n
        var = s_ref[1:2, :] * inv_n - mean * mean
        sc = p_ref[0:1, :] * lax.rsqrt(var + _BN_EPS)
        tr = p_ref[1:2, :] - mean * sc
        zn = z_ref[...] * sc + tr
        zn = jnp.where(zn >= 0, zn, 0.2 * zn)
        acc = jnp.dot(zn, wa_ref[...], preferred_element_type=jnp.float32)
        acc = acc + jnp.dot(x1_ref[...], wb_ref[...],
                            preferred_element_type=jnp.float32)
        o_ref[...] = acc + p_ref[2:3, :]

    return pl.pallas_call(
        body,
        grid=(G,),
        in_specs=[pl.BlockSpec((_TM, C), lambda i: (i, 0)),
                  pl.BlockSpec((_TM, C), lambda i: (i, 0)),
                  pl.BlockSpec((8, C), lambda i: (0, 0)),
                  pl.BlockSpec((8, C), lambda i: (0, 0)),
                  pl.BlockSpec((C, C), lambda i: (0, 0)),
                  pl.BlockSpec((C, C), lambda i: (0, 0))],
        out_specs=pl.BlockSpec((_TM, C), lambda i: (i, 0)),
        out_shape=jax.ShapeDtypeStruct((P, C), jnp.float32),
    )(z, x1p, stats, params, WaT, WbT)


def kernel(x, x1, neigh_orders, pool_neigh_orders, W1, b1, gamma, beta, Wc, bc):
    B, C, Nh = x.shape
    CO = x1.shape[1]
    Nl = (Nh + 6) // 4
    R = B * Nl
    P = ((R + _NW * 16 - 1) // (_NW * 16)) * (_NW * 16)

    xT = x.transpose(0, 2, 1).reshape(B * Nh, C)
    x1p = jnp.pad(x1.transpose(0, 2, 1).reshape(R, CO), ((0, P - R), (0, 0)))

    boffs_h = (jnp.arange(B, dtype=jnp.int32) * Nh)[:, None]
    pool_abs = (pool_neigh_orders[: Nl * 7][None, :] + boffs_h).reshape(-1)
    pool_abs = jnp.pad(pool_abs, (0, (P - R) * 7))

    boffs_l = (jnp.arange(B, dtype=jnp.int32) * Nl)[:, None]
    k_off = jnp.tile(jnp.arange(7, dtype=jnp.int32), Nl)[None, :]
    ring_abs = ((neigh_orders[None, :] + boffs_l) * 7 + k_off).reshape(-1)
    ring_abs = jnp.pad(ring_abs, (0, (P - R) * 7))

    Wz = W1.reshape(CO, 7, C).transpose(2, 1, 0).reshape(C, 7 * CO)
    WaT = Wc[:, :CO].T
    WbT = Wc[:, CO:].T
    params = jnp.concatenate(
        [gamma[None, :], beta[None, :], bc[None, :],
         jnp.zeros((5, CO), jnp.float32)], axis=0)

    idx_shape = (_NW, P // (16 * _NW), 112)
    xp = _gather_sum7_sc(xT, pool_abs.reshape(idx_shape), P)
    zp = _zp_matmul_tc(xp, Wz, 1.0 / 7.0)
    z = _gather_sum7_sc(zp.reshape(P * 7, CO),
                        ring_abs.reshape(idx_shape), P)
    stats = _stats_tc(z, R)
    yT = _final_tc(z, x1p, stats, params, WaT, WbT, R)
    return yT[:R].reshape(B, Nl, CO).transpose(0, 2, 1)
